# Initial kernel scaffold; baseline (speedup 1.0000x reference)
import jax
import jax.numpy as jnp
from jax.experimental import pallas as pl


def kernel(x, W1, b1, g1, be1, W2, b2, g2, be2, edge_index):
    raise NotImplementedError("write your pallas kernel here")


# speedup vs baseline: 1.7028x; 1.7028x over previous
import functools

import jax
import jax.numpy as jnp
from jax import lax
from jax.experimental import pallas as pl
from jax.experimental.pallas import tpu as pltpu
from jax.experimental.pallas import tpu_sc as plsc

F32 = jnp.float32

SC_CORES = 2
SC_SUBCORES = 16
NW = SC_CORES * SC_SUBCORES

GATHER_CHUNK = 80
SCATTER_CHUNK = 80
CNT_W = 16


def _uv_kernel(x_ref, wt_ref, w_ref):
    w_ref[...] = jnp.dot(x_ref[...], wt_ref[...],
                         preferred_element_type=F32)


def _stats1_kernel(gu_ref, gv_ref, s_ref, acc_ref):
    i = pl.program_id(0)

    @pl.when(i == 0)
    def _():
        acc_ref[...] = jnp.zeros_like(acc_ref)

    h = gu_ref[...] + gv_ref[...]
    acc_ref[0:1, :] += jnp.sum(h, axis=0, keepdims=True)
    acc_ref[1:2, :] += jnp.sum(h * h, axis=0, keepdims=True)

    @pl.when(i == pl.num_programs(0) - 1)
    def _():
        s_ref[...] = acc_ref[...]


def _mlp2_kernel(gu_ref, gv_ref, s1_ref, g1_ref, be1_ref, w2t_ref,
                 h2_ref, s2_ref, acc_ref, *, inv_e):
    i = pl.program_id(0)

    @pl.when(i == 0)
    def _():
        acc_ref[...] = jnp.zeros_like(acc_ref)

    mean1 = s1_ref[0:1, :] * inv_e
    var1 = s1_ref[1:2, :] * inv_e - mean1 * mean1
    a1 = g1_ref[...] * lax.rsqrt(var1 + 1e-5)
    c1 = be1_ref[...] - mean1 * a1

    h1 = gu_ref[...] + gv_ref[...]
    z = jnp.maximum(h1 * a1 + c1, 0.0)
    h2 = jnp.dot(z, w2t_ref[...], preferred_element_type=F32)
    h2_ref[...] = h2
    acc_ref[0:1, :] += jnp.sum(h2, axis=0, keepdims=True)
    acc_ref[1:2, :] += jnp.sum(h2 * h2, axis=0, keepdims=True)

    @pl.when(i == pl.num_programs(0) - 1)
    def _():
        s2_ref[...] = acc_ref[...]


def _msg_kernel(h2_ref, s2_ref, g2_ref, be2_ref, msg_ref, *, inv_e):
    mean2 = s2_ref[0:1, :] * inv_e
    var2 = s2_ref[1:2, :] * inv_e - mean2 * mean2
    a2 = g2_ref[...] * lax.rsqrt(var2 + 1e-5)
    c2 = be2_ref[...] - mean2 * a2
    msg_ref[...] = jnp.maximum(h2_ref[...] * a2 + c2, 0.0)


def _combine_kernel(macc_ref, cacc_ref, out_ref):
    cnt = cacc_ref[0, :, 0:1]
    out_ref[...] = macc_ref[0] / jnp.maximum(cnt, 1.0)


def kernel(x, W1, b1, g1, be1, W2, b2, g2, be2, edge_index):
    n, d = x.shape
    h = W1.shape[0]
    e = edge_index.shape[1]
    per_w = e // NW
    half = 5120
    acc_rows = 5248
    rows_per_sub = acc_rows // SC_SUBCORES

    src = edge_index[0]
    dst = edge_index[1]
    src2 = src + jnp.int32(n)

    w1t = W1.T
    w2t = W2.T
    g1r = g1.reshape(1, h)
    be1r = be1.reshape(1, h)
    g2r = g2.reshape(1, h)
    be2r = be2.reshape(1, h)

    bn_rows = 1000
    nb = n // bn_rows
    w = pl.pallas_call(
        _uv_kernel,
        grid=(2, nb),
        in_specs=[
            pl.BlockSpec((bn_rows, d), lambda p, i: (i, 0)),
            pl.BlockSpec((d, h), lambda p, i: (p, 0)),
        ],
        out_specs=pl.BlockSpec((bn_rows, h), lambda p, i: (p * nb + i, 0)),
        out_shape=jax.ShapeDtypeStruct((2 * n, h), F32),
    )(x, w1t)

    mesh = plsc.VectorSubcoreMesh(core_axis_name="c", subcore_axis_name="s")
    c_g = GATHER_CHUNK

    @functools.partial(
        pl.kernel,
        mesh=mesh,
        out_type=[jax.ShapeDtypeStruct((e, h), F32),
                  jax.ShapeDtypeStruct((e, h), F32)],
        scratch_types=[
            pltpu.VMEM((c_g,), jnp.int32),
            pltpu.VMEM((c_g,), jnp.int32),
            pltpu.VMEM((c_g, h), F32),
            pltpu.VMEM((c_g, h), F32),
            pltpu.SemaphoreType.DMA,
            pltpu.SemaphoreType.DMA,
        ],
    )
    def _gather_kernel(w_hbm, dst_hbm, src2_hbm, gu_hbm, gv_hbm,
                       idxd, idxs, bu, bv, sem1, sem2):
        wid = lax.axis_index("s") * SC_CORES + lax.axis_index("c")
        base = wid * per_w

        @pl.loop(0, per_w, step=c_g)
        def _(off):
            b = base + off
            pltpu.sync_copy(dst_hbm.at[pl.ds(b, c_g)], idxd)
            pltpu.sync_copy(src2_hbm.at[pl.ds(b, c_g)], idxs)
            cp1 = pltpu.async_copy(w_hbm.at[idxd], bu, sem1)
            cp2 = pltpu.async_copy(w_hbm.at[idxs], bv, sem2)
            cp1.wait()
            cp2.wait()
            pltpu.sync_copy(bu, gu_hbm.at[pl.ds(b, c_g)])
            pltpu.sync_copy(bv, gv_hbm.at[pl.ds(b, c_g)])

    gu, gv = _gather_kernel(w, dst, src2)

    be_rows = 2000
    ge = e // be_rows
    s1 = pl.pallas_call(
        _stats1_kernel,
        grid=(ge,),
        in_specs=[
            pl.BlockSpec((be_rows, h), lambda i: (i, 0)),
            pl.BlockSpec((be_rows, h), lambda i: (i, 0)),
        ],
        out_specs=pl.BlockSpec((2, h), lambda i: (0, 0)),
        out_shape=jax.ShapeDtypeStruct((2, h), F32),
        scratch_shapes=[pltpu.VMEM((2, h), F32)],
    )(gu, gv)

    h2, s2 = pl.pallas_call(
        functools.partial(_mlp2_kernel, inv_e=1.0 / e),
        grid=(ge,),
        in_specs=[
            pl.BlockSpec((be_rows, h), lambda i: (i, 0)),
            pl.BlockSpec((be_rows, h), lambda i: (i, 0)),
            pl.BlockSpec((2, h), lambda i: (0, 0)),
            pl.BlockSpec((1, h), lambda i: (0, 0)),
            pl.BlockSpec((1, h), lambda i: (0, 0)),
            pl.BlockSpec((h, h), lambda i: (0, 0)),
        ],
        out_specs=[
            pl.BlockSpec((be_rows, h), lambda i: (i, 0)),
            pl.BlockSpec((2, h), lambda i: (0, 0)),
        ],
        out_shape=[jax.ShapeDtypeStruct((e, h), F32),
                   jax.ShapeDtypeStruct((2, h), F32)],
        scratch_shapes=[pltpu.VMEM((2, h), F32)],
    )(gu, gv, s1, g1r, be1r, w2t)

    msg = pl.pallas_call(
        functools.partial(_msg_kernel, inv_e=1.0 / e),
        grid=(ge,),
        in_specs=[
            pl.BlockSpec((be_rows, h), lambda i: (i, 0)),
            pl.BlockSpec((2, h), lambda i: (0, 0)),
            pl.BlockSpec((1, h), lambda i: (0, 0)),
            pl.BlockSpec((1, h), lambda i: (0, 0)),
        ],
        out_specs=pl.BlockSpec((be_rows, h), lambda i: (i, 0)),
        out_shape=jax.ShapeDtypeStruct((e, h), F32),
    )(h2, s2, g2r, be2r)

    per_w_s = e // SC_SUBCORES
    c_s = SCATTER_CHUNK
    ones_rows = jnp.zeros((c_s, CNT_W), F32).at[:, 0].set(1.0)
    z_rows = jnp.zeros((rows_per_sub, h), F32)

    @functools.partial(
        pl.kernel,
        mesh=mesh,
        out_type=jax.ShapeDtypeStruct((SC_CORES, acc_rows, h), F32),
        scratch_types=[
            pltpu.VMEM((c_s, h), F32),
            pltpu.VMEM((c_s,), jnp.int32),
            pltpu.VMEM((c_s,), jnp.int32),
            pltpu.VMEM_SHARED((acc_rows, h), F32),
        ],
    )
    def _scatter_kernel(msg_hbm, dst_hbm, z128_hbm, macc_hbm,
                        buf, idx, idxt, sacc):
        cid = lax.axis_index("c")
        sid = lax.axis_index("s")
        my_rows = pl.ds(sid * rows_per_sub, rows_per_sub)
        pltpu.sync_copy(z128_hbm, sacc.at[my_rows])
        plsc.subcore_barrier()

        base = sid * per_w_s
        row0 = cid * half

        @pl.loop(0, per_w_s, step=c_s)
        def _(off):
            b = base + off
            pltpu.sync_copy(dst_hbm.at[pl.ds(b, c_s)], idx)
            pltpu.sync_copy(msg_hbm.at[pl.ds(b, c_s)], buf)

            @pl.loop(0, c_s, step=16)
            def _(i):
                t = idx[pl.ds(i, 16)] - row0
                keep = (t >= 0) & (t < half)
                idxt[pl.ds(i, 16)] = jnp.where(keep, t, half)

            pltpu.sync_copy(buf, sacc.at[idxt], add=True)

        plsc.subcore_barrier()
        pltpu.sync_copy(sacc.at[my_rows], macc_hbm.at[cid, my_rows])

    macc = _scatter_kernel(msg, dst, z_rows)

    ones_rows = jnp.zeros((c_s, h), F32).at[:, 0].set(1.0)

    @functools.partial(
        pl.kernel,
        mesh=mesh,
        out_type=jax.ShapeDtypeStruct((SC_CORES, acc_rows, h), F32),
        scratch_types=[
            pltpu.VMEM((c_s, h), F32),
            pltpu.VMEM((c_s,), jnp.int32),
            pltpu.VMEM((c_s,), jnp.int32),
            pltpu.VMEM_SHARED((acc_rows, h), F32),
        ],
    )
    def _count_kernel(dst_hbm, ones_hbm, z128_hbm, cacc_hbm,
                      ones, idx, idxt, scnt):
        cid = lax.axis_index("c")
        sid = lax.axis_index("s")
        my_rows = pl.ds(sid * rows_per_sub, rows_per_sub)
        pltpu.sync_copy(z128_hbm, scnt.at[my_rows])
        pltpu.sync_copy(ones_hbm, ones)
        plsc.subcore_barrier()

        base = sid * per_w_s
        row0 = cid * half

        @pl.loop(0, per_w_s, step=c_s)
        def _(off):
            b = base + off
            pltpu.sync_copy(dst_hbm.at[pl.ds(b, c_s)], idx)
            pltpu.sync_copy(ones_hbm, ones)

            @pl.loop(0, c_s, step=16)
            def _(i):
                t = idx[pl.ds(i, 16)] - row0
                keep = (t >= 0) & (t < half)
                idxt[pl.ds(i, 16)] = jnp.where(keep, t, half)

            pltpu.sync_copy(ones, scnt.at[idxt], add=True)

        plsc.subcore_barrier()
        pltpu.sync_copy(scnt.at[my_rows], cacc_hbm.at[cid, my_rows])

    cacc = _count_kernel(dst, ones_rows, z_rows)

    bn_out = 1280
    out_pad = pl.pallas_call(
        _combine_kernel,
        grid=(SC_CORES, half // bn_out),
        in_specs=[
            pl.BlockSpec((1, bn_out, h), lambda c, i: (c, i, 0)),
            pl.BlockSpec((1, bn_out, h), lambda c, i: (c, i, 0)),
        ],
        out_specs=pl.BlockSpec((bn_out, h),
                               lambda c, i: (c * (half // bn_out) + i, 0)),
        out_shape=jax.ShapeDtypeStruct((SC_CORES * half, h), F32),
    )(macc, cacc)
    out = out_pad[:n]

    return (out, msg)


# speedup vs baseline: 2.3253x; 1.3656x over previous
import functools

import jax
import jax.numpy as jnp
from jax import lax
from jax.experimental import pallas as pl
from jax.experimental.pallas import tpu as pltpu
from jax.experimental.pallas import tpu_sc as plsc

F32 = jnp.float32

SC_CORES = 2
SC_SUBCORES = 16
NW = SC_CORES * SC_SUBCORES

GATHER_CHUNK = 80
SCATTER_CHUNK = 80
CNT_W = 16


def _uv_kernel(x_ref, wt_ref, w_ref):
    w_ref[...] = jnp.dot(x_ref[...], wt_ref[...],
                         preferred_element_type=F32)


def _stats1_kernel(gu_ref, gv_ref, s_ref, acc_ref):
    i = pl.program_id(0)

    @pl.when(i == 0)
    def _():
        acc_ref[...] = jnp.zeros_like(acc_ref)

    h = gu_ref[...] + gv_ref[...]
    acc_ref[0:1, :] += jnp.sum(h, axis=0, keepdims=True)
    acc_ref[1:2, :] += jnp.sum(h * h, axis=0, keepdims=True)

    @pl.when(i == pl.num_programs(0) - 1)
    def _():
        s_ref[...] = acc_ref[...]


def _mlp2_kernel(gu_ref, gv_ref, s1_ref, g1_ref, be1_ref, w2t_ref,
                 h2_ref, s2_ref, acc_ref, *, inv_e):
    i = pl.program_id(0)

    @pl.when(i == 0)
    def _():
        acc_ref[...] = jnp.zeros_like(acc_ref)

    mean1 = s1_ref[0:1, :] * inv_e
    var1 = s1_ref[1:2, :] * inv_e - mean1 * mean1
    a1 = g1_ref[...] * lax.rsqrt(var1 + 1e-5)
    c1 = be1_ref[...] - mean1 * a1

    h1 = gu_ref[...] + gv_ref[...]
    z = jnp.maximum(h1 * a1 + c1, 0.0)
    h2 = jnp.dot(z, w2t_ref[...], preferred_element_type=F32)
    h2_ref[...] = h2
    acc_ref[0:1, :] += jnp.sum(h2, axis=0, keepdims=True)
    acc_ref[1:2, :] += jnp.sum(h2 * h2, axis=0, keepdims=True)

    @pl.when(i == pl.num_programs(0) - 1)
    def _():
        s2_ref[...] = acc_ref[...]


def _msg_kernel(h2_ref, s2_ref, g2_ref, be2_ref, msg_ref, *, inv_e):
    mean2 = s2_ref[0:1, :] * inv_e
    var2 = s2_ref[1:2, :] * inv_e - mean2 * mean2
    a2 = g2_ref[...] * lax.rsqrt(var2 + 1e-5)
    c2 = be2_ref[...] - mean2 * a2
    msg_ref[...] = jnp.maximum(h2_ref[...] * a2 + c2, 0.0)


def _combine_kernel(macc_ref, cacc_ref, out_ref):
    cnt = cacc_ref[0, :, 0:1]
    out_ref[...] = macc_ref[0] / jnp.maximum(cnt, 1.0)


def kernel(x, W1, b1, g1, be1, W2, b2, g2, be2, edge_index):
    n, d = x.shape
    h = W1.shape[0]
    e = edge_index.shape[1]
    per_w = e // NW
    half = 5120
    acc_rows = 5248
    rows_per_sub = acc_rows // SC_SUBCORES

    src = edge_index[0]
    dst = edge_index[1]
    src2 = src + jnp.int32(n)

    w1t = W1.T
    w2t = W2.T
    g1r = g1.reshape(1, h)
    be1r = be1.reshape(1, h)
    g2r = g2.reshape(1, h)
    be2r = be2.reshape(1, h)

    bn_rows = 1000
    nb = n // bn_rows
    w = pl.pallas_call(
        _uv_kernel,
        grid=(2, nb),
        in_specs=[
            pl.BlockSpec((bn_rows, d), lambda p, i: (i, 0)),
            pl.BlockSpec((d, h), lambda p, i: (p, 0)),
        ],
        out_specs=pl.BlockSpec((bn_rows, h), lambda p, i: (p * nb + i, 0)),
        out_shape=jax.ShapeDtypeStruct((2 * n, h), F32),
    )(x, w1t)

    mesh = plsc.VectorSubcoreMesh(core_axis_name="c", subcore_axis_name="s")
    c_g = GATHER_CHUNK

    @functools.partial(
        pl.kernel,
        mesh=mesh,
        out_type=[jax.ShapeDtypeStruct((e, h), F32),
                  jax.ShapeDtypeStruct((e, h), F32)],
        scratch_types=[
            pltpu.VMEM((c_g,), jnp.int32),
            pltpu.VMEM((c_g,), jnp.int32),
            pltpu.VMEM((c_g, h), F32),
            pltpu.VMEM((c_g, h), F32),
            pltpu.SemaphoreType.DMA,
            pltpu.SemaphoreType.DMA,
        ],
    )
    def _gather_kernel(w_hbm, dst_hbm, src2_hbm, gu_hbm, gv_hbm,
                       idxd, idxs, bu, bv, sem1, sem2):
        wid = lax.axis_index("s") * SC_CORES + lax.axis_index("c")
        base = wid * per_w

        @pl.loop(0, per_w, step=c_g)
        def _(off):
            b = base + off
            pltpu.sync_copy(dst_hbm.at[pl.ds(b, c_g)], idxd)
            pltpu.sync_copy(src2_hbm.at[pl.ds(b, c_g)], idxs)
            cp1 = pltpu.async_copy(w_hbm.at[idxd], bu, sem1)
            cp2 = pltpu.async_copy(w_hbm.at[idxs], bv, sem2)
            cp1.wait()
            cp2.wait()
            pltpu.sync_copy(bu, gu_hbm.at[pl.ds(b, c_g)])
            pltpu.sync_copy(bv, gv_hbm.at[pl.ds(b, c_g)])

    gu, gv = _gather_kernel(w, dst, src2)

    per_w_s = e // SC_SUBCORES
    c_s = SCATTER_CHUNK
    ones_rows = jnp.zeros((c_s, h), F32).at[:, 0].set(1.0)
    z_rows = jnp.zeros((rows_per_sub, h), F32)

    @functools.partial(
        pl.kernel,
        mesh=mesh,
        out_type=jax.ShapeDtypeStruct((SC_CORES, acc_rows, h), F32),
        scratch_types=[
            pltpu.VMEM((c_s, h), F32),
            pltpu.VMEM((c_s,), jnp.int32),
            pltpu.VMEM((c_s,), jnp.int32),
            pltpu.VMEM_SHARED((acc_rows, h), F32),
            pltpu.VMEM_SHARED((c_s, h), F32),
        ],
    )
    def _count_kernel(dst_hbm, ones_hbm, z128_hbm, cacc_hbm,
                      ones, idx, idxt, scnt, tmpl):
        cid = lax.axis_index("c")
        sid = lax.axis_index("s")
        my_rows = pl.ds(sid * rows_per_sub, rows_per_sub)
        pltpu.sync_copy(z128_hbm, scnt.at[my_rows])

        @pl.when(sid == 0)
        def _():
            pltpu.sync_copy(ones_hbm, tmpl)

        plsc.subcore_barrier()

        base = sid * per_w_s
        row0 = cid * half

        @pl.loop(0, per_w_s, step=c_s)
        def _(off):
            b = base + off
            pltpu.sync_copy(dst_hbm.at[pl.ds(b, c_s)], idx)
            pltpu.sync_copy(tmpl, ones)

            @pl.loop(0, c_s, step=16)
            def _(i):
                t = idx[pl.ds(i, 16)] - row0
                keep = (t >= 0) & (t < half)
                idxt[pl.ds(i, 16)] = jnp.where(keep, t, half)

            pltpu.sync_copy(ones, scnt.at[idxt], add=True)

        plsc.subcore_barrier()
        pltpu.sync_copy(scnt.at[my_rows], cacc_hbm.at[cid, my_rows])

    cacc = _count_kernel(dst, ones_rows, z_rows)

    be_rows = 2000
    ge = e // be_rows
    s1 = pl.pallas_call(
        _stats1_kernel,
        grid=(ge,),
        in_specs=[
            pl.BlockSpec((be_rows, h), lambda i: (i, 0)),
            pl.BlockSpec((be_rows, h), lambda i: (i, 0)),
        ],
        out_specs=pl.BlockSpec((2, h), lambda i: (0, 0)),
        out_shape=jax.ShapeDtypeStruct((2, h), F32),
        scratch_shapes=[pltpu.VMEM((2, h), F32)],
    )(gu, gv)

    h2, s2 = pl.pallas_call(
        functools.partial(_mlp2_kernel, inv_e=1.0 / e),
        grid=(ge,),
        in_specs=[
            pl.BlockSpec((be_rows, h), lambda i: (i, 0)),
            pl.BlockSpec((be_rows, h), lambda i: (i, 0)),
            pl.BlockSpec((2, h), lambda i: (0, 0)),
            pl.BlockSpec((1, h), lambda i: (0, 0)),
            pl.BlockSpec((1, h), lambda i: (0, 0)),
            pl.BlockSpec((h, h), lambda i: (0, 0)),
        ],
        out_specs=[
            pl.BlockSpec((be_rows, h), lambda i: (i, 0)),
            pl.BlockSpec((2, h), lambda i: (0, 0)),
        ],
        out_shape=[jax.ShapeDtypeStruct((e, h), F32),
                   jax.ShapeDtypeStruct((2, h), F32)],
        scratch_shapes=[pltpu.VMEM((2, h), F32)],
    )(gu, gv, s1, g1r, be1r, w2t)

    msg = pl.pallas_call(
        functools.partial(_msg_kernel, inv_e=1.0 / e),
        grid=(ge,),
        in_specs=[
            pl.BlockSpec((be_rows, h), lambda i: (i, 0)),
            pl.BlockSpec((2, h), lambda i: (0, 0)),
            pl.BlockSpec((1, h), lambda i: (0, 0)),
            pl.BlockSpec((1, h), lambda i: (0, 0)),
        ],
        out_specs=pl.BlockSpec((be_rows, h), lambda i: (i, 0)),
        out_shape=jax.ShapeDtypeStruct((e, h), F32),
    )(h2, s2, g2r, be2r)

    @functools.partial(
        pl.kernel,
        mesh=mesh,
        out_type=jax.ShapeDtypeStruct((SC_CORES, acc_rows, h), F32),
        scratch_types=[
            pltpu.VMEM((c_s, h), F32),
            pltpu.VMEM((c_s,), jnp.int32),
            pltpu.VMEM((c_s,), jnp.int32),
            pltpu.VMEM_SHARED((acc_rows, h), F32),
        ],
    )
    def _scatter_kernel(msg_hbm, dst_hbm, z128_hbm, macc_hbm,
                        buf, idx, idxt, sacc):
        cid = lax.axis_index("c")
        sid = lax.axis_index("s")
        my_rows = pl.ds(sid * rows_per_sub, rows_per_sub)
        pltpu.sync_copy(z128_hbm, sacc.at[my_rows])
        plsc.subcore_barrier()

        base = sid * per_w_s
        row0 = cid * half

        @pl.loop(0, per_w_s, step=c_s)
        def _(off):
            b = base + off
            pltpu.sync_copy(dst_hbm.at[pl.ds(b, c_s)], idx)
            pltpu.sync_copy(msg_hbm.at[pl.ds(b, c_s)], buf)

            @pl.loop(0, c_s, step=16)
            def _(i):
                t = idx[pl.ds(i, 16)] - row0
                keep = (t >= 0) & (t < half)
                idxt[pl.ds(i, 16)] = jnp.where(keep, t, half)

            pltpu.sync_copy(buf, sacc.at[idxt], add=True)

        plsc.subcore_barrier()
        pltpu.sync_copy(sacc.at[my_rows], macc_hbm.at[cid, my_rows])

    macc = _scatter_kernel(msg, dst, z_rows)

    bn_out = 1280
    out_pad = pl.pallas_call(
        _combine_kernel,
        grid=(SC_CORES, half // bn_out),
        in_specs=[
            pl.BlockSpec((1, bn_out, h), lambda c, i: (c, i, 0)),
            pl.BlockSpec((1, bn_out, h), lambda c, i: (c, i, 0)),
        ],
        out_specs=pl.BlockSpec((bn_out, h),
                               lambda c, i: (c * (half // bn_out) + i, 0)),
        out_shape=jax.ShapeDtypeStruct((SC_CORES * half, h), F32),
    )(macc, cacc)
    out = out_pad[:n]

    return (out, msg)
